# Initial kernel scaffold; baseline (speedup 1.0000x reference)
import jax
import jax.numpy as jnp
from jax.experimental import pallas as pl


def kernel(node_features, col_features, edge_index, W1, b1, W2, b2, Wfc, bfc, Wc1, bc1, Wc2, bc2):
    raise NotImplementedError("write your pallas kernel here")


# speedup vs baseline: 146.9583x; 146.9583x over previous
import functools

import jax
import jax.numpy as jnp
from jax import lax
from jax.experimental import pallas as pl
from jax.experimental.pallas import tpu as pltpu
from jax.experimental.pallas import tpu_sc as plsc

F32 = jnp.float32

_B, _N, _F, _K, _CF, _E = 8, 10000, 128, 8, 16, 320000
_NPAD = 10016
_NW = 32
_GRP = 128
_GP = 79
_EPAD = _NW * _GP * _GRP


def _sc_mesh():
    return plsc.VectorSubcoreMesh(core_axis_name="c", subcore_axis_name="s")


_SC_PARAMS = pltpu.CompilerParams(use_tc_tiling_on_sc=False)


def _sc_deg(dstp, ones_g, zeros_np):
    @functools.partial(
        pl.kernel,
        out_type=jax.ShapeDtypeStruct((2, _NPAD, 16), F32),
        mesh=_sc_mesh(),
        compiler_params=_SC_PARAMS,
        scratch_types=[
            pltpu.VMEM((_GP, _GRP), jnp.int32),
            pltpu.VMEM((_GRP, 16), F32),
            pltpu.VMEM_SHARED((_NPAD, 16), F32),
        ],
    )
    def k(dst_hbm, ones_hbm, zeros_hbm, out_hbm, didx_v, ones_v, shared):
        cid = lax.axis_index("c")
        sid = lax.axis_index("s")
        wid = sid * 2 + cid
        pltpu.sync_copy(dst_hbm.at[wid], didx_v)
        pltpu.sync_copy(ones_hbm, ones_v)

        @pl.when(sid == 0)
        def _():
            pltpu.sync_copy(zeros_hbm, shared)

        plsc.subcore_barrier()

        def body(j, carry):
            pltpu.sync_copy(ones_v, shared.at[didx_v.at[j]], add=True)
            return carry

        lax.fori_loop(0, _GP, body, 0)
        plsc.subcore_barrier()

        @pl.when(sid == 0)
        def _():
            pltpu.sync_copy(shared, out_hbm.at[cid])

    return k(dstp, ones_g, zeros_np)


def _sc_agg(g, srcp, dstp, zeros_np):
    @functools.partial(
        pl.kernel,
        out_type=jax.ShapeDtypeStruct((2, _NPAD, 16), F32),
        mesh=_sc_mesh(),
        compiler_params=_SC_PARAMS,
        scratch_types=[
            pltpu.VMEM((_GP, _GRP), jnp.int32),
            pltpu.VMEM((_GP, _GRP), jnp.int32),
            pltpu.VMEM((_GRP, 16), F32),
            pltpu.VMEM_SHARED((_NPAD, 16), F32),
            pltpu.SemaphoreType.DMA,
        ],
    )
    def k(g_hbm, src_hbm, dst_hbm, zeros_hbm, out_hbm,
          sidx_v, didx_v, buf, shared, sem):
        cid = lax.axis_index("c")
        sid = lax.axis_index("s")
        wid = sid * 2 + cid
        pltpu.sync_copy(src_hbm.at[wid], sidx_v)
        pltpu.sync_copy(dst_hbm.at[wid], didx_v)

        @pl.when(sid == 0)
        def _():
            pltpu.sync_copy(zeros_hbm, shared)

        plsc.subcore_barrier()

        def body(j, carry):
            pltpu.async_copy(g_hbm.at[sidx_v.at[j]], buf, sem).wait()
            pltpu.sync_copy(buf, shared.at[didx_v.at[j]], add=True)
            return carry

        lax.fori_loop(0, _GP, body, 0)
        plsc.subcore_barrier()

        @pl.when(sid == 0)
        def _():
            pltpu.sync_copy(shared, out_hbm.at[cid])

    return k(g, srcp, dstp, zeros_np)


def _tc_a1(x0p, w1):
    def body(x_ref, w_ref, o_ref):
        o_ref[...] = jnp.dot(x_ref[...], w_ref[...],
                             preferred_element_type=F32)

    return pl.pallas_call(
        body, out_shape=jax.ShapeDtypeStruct((_NPAD, 16), F32))(x0p, w1)


def _tc_prep(degp, p0):
    def body(d_ref, p_ref, dinv_ref, g1_ref):
        cnt = d_ref[0] + d_ref[1]
        deg = cnt * float(_B) + 1.0
        dinv = lax.rsqrt(deg)
        dinv_ref[...] = dinv
        g1_ref[...] = p_ref[...] * dinv

    return pl.pallas_call(
        body,
        out_shape=(jax.ShapeDtypeStruct((_NPAD, 16), F32),
                   jax.ShapeDtypeStruct((_NPAD, 16), F32)))(degp, p0)


def _tc_comb1(s1p, p0, dinv16, b1r, w2):
    def body(s_ref, p_ref, dinv_ref, b1_ref, w2_ref, q0_ref, g2_ref):
        dinv = dinv_ref[...]
        s = s_ref[0] + s_ref[1]
        h1 = jnp.maximum(
            s * (dinv * float(_B)) + dinv * dinv * p_ref[...] + b1_ref[...],
            0.0)
        q0 = jnp.dot(h1, w2_ref[...], preferred_element_type=F32)
        q0_ref[...] = q0
        g2_ref[...] = q0 * dinv

    return pl.pallas_call(
        body,
        out_shape=(jax.ShapeDtypeStruct((_NPAD, 16), F32),
                   jax.ShapeDtypeStruct((_NPAD, 16), F32)))(
            s1p, p0, dinv16, b1r, w2)


def _tc_comb2(s2p, q0, dinv16, b2r, wfct, bfcr):
    def body(s_ref, q_ref, dinv_ref, b2_ref, wt_ref, bfc_ref, o_ref):
        dinv = dinv_ref[...]
        s = s_ref[0] + s_ref[1]
        h2 = jnp.maximum(
            s * (dinv * float(_B)) + dinv * dinv * q_ref[...] + b2_ref[...],
            0.0)
        o_ref[...] = lax.dot_general(
            wt_ref[...], h2, (((1,), (1,)), ((), ())),
            preferred_element_type=F32) + bfc_ref[...]

    return pl.pallas_call(
        body,
        out_shape=jax.ShapeDtypeStruct((1, _NPAD), F32))(
            s2p, q0, dinv16, b2r, wfct, bfcr)


_RBLK = 2800


def _tc_mlp_rest(xr, w1, b1r, w2, b2r, wfct, bfcr):
    nblk = xr.shape[0] // _RBLK

    def body(x_ref, w1_ref, b1_ref, w2_ref, b2_ref, wt_ref, bfc_ref, o_ref):
        p = jnp.dot(x_ref[...], w1_ref[...], preferred_element_type=F32)
        h1 = jnp.maximum(p + b1_ref[...], 0.0)
        h2 = jnp.maximum(
            jnp.dot(h1, w2_ref[...], preferred_element_type=F32)
            + b2_ref[...], 0.0)
        o_ref[0] = lax.dot_general(
            wt_ref[...], h2, (((1,), (1,)), ((), ())),
            preferred_element_type=F32) + bfc_ref[...]

    return pl.pallas_call(
        body,
        grid=(nblk,),
        in_specs=[
            pl.BlockSpec((_RBLK, _F), lambda i: (i, 0)),
            pl.BlockSpec((_F, 16), lambda i: (0, 0)),
            pl.BlockSpec((1, 16), lambda i: (0, 0)),
            pl.BlockSpec((16, 16), lambda i: (0, 0)),
            pl.BlockSpec((1, 16), lambda i: (0, 0)),
            pl.BlockSpec((1, 16), lambda i: (0, 0)),
            pl.BlockSpec((1, 1), lambda i: (0, 0)),
        ],
        out_specs=pl.BlockSpec((1, 1, _RBLK), lambda i: (i, 0, 0)),
        out_shape=jax.ShapeDtypeStruct((nblk, 1, _RBLK), F32),
    )(xr, w1, b1r, w2, b2r, wfct, bfcr)


_CBLK = 3200


def _tc_col(cols2, bd1, bc1t, bd2t, bc2r):
    nblk = cols2.shape[0] // _CBLK

    def body(c_ref, bd1_ref, bc1_ref, bd2t_ref, bc2_ref, o_ref):
        h = jnp.maximum(
            jnp.dot(c_ref[...], bd1_ref[...], preferred_element_type=F32)
            + bc1_ref[...], 0.0)
        ch = lax.dot_general(
            bd2t_ref[...], h, (((1,), (1,)), ((), ())),
            preferred_element_type=F32) + bc2_ref[...]
        m = jnp.max(ch, axis=0, keepdims=True)
        e = jnp.exp(ch - m)
        o_ref[...] = e / jnp.sum(e, axis=0, keepdims=True)

    return pl.pallas_call(
        body,
        grid=(nblk,),
        in_specs=[
            pl.BlockSpec((_CBLK, _K * _CF), lambda i: (i, 0)),
            pl.BlockSpec((_K * _CF, _K * _CF), lambda i: (0, 0)),
            pl.BlockSpec((1, _K * _CF), lambda i: (0, 0)),
            pl.BlockSpec((_K, _K * _CF), lambda i: (0, 0)),
            pl.BlockSpec((1, 1), lambda i: (0, 0)),
        ],
        out_specs=pl.BlockSpec((_K, _CBLK), lambda i: (0, i)),
        out_shape=jax.ShapeDtypeStruct((_K, _B * _N), F32),
    )(cols2, bd1, bc1t, bd2t, bc2r)


def _tc_softprod(node_out, colp3):
    def body(no_ref, c_ref, o_ref):
        x = no_ref[...]
        m = jnp.max(x, axis=1, keepdims=True)
        e = jnp.exp(x - m)
        p = e / jnp.sum(e, axis=1, keepdims=True)
        o_ref[...] = c_ref[...] * p[None]

    return pl.pallas_call(
        body,
        out_shape=jax.ShapeDtypeStruct((_K, _B, _N), F32))(node_out, colp3)


def kernel(node_features, col_features, edge_index, W1, b1, W2, b2,
           Wfc, bfc, Wc1, bc1, Wc2, bc2):
    b, n, f = node_features.shape
    assert (b, n, f) == (_B, _N, _F)
    e = edge_index.shape[1]
    assert e == _E

    x0p = jnp.pad(node_features[0], ((0, _NPAD - _N), (0, 0)))
    xr = node_features[1:].reshape((b - 1) * n, f)
    cols2 = col_features.reshape(b * n, _K * _CF)

    pad_e = _EPAD - _E
    pad_idx = jnp.full((pad_e,), _N, dtype=edge_index.dtype)
    srcp = jnp.concatenate([edge_index[0], pad_idx]).reshape(_NW, _GP, _GRP)
    dstp = jnp.concatenate([edge_index[1], pad_idx]).reshape(_NW, _GP, _GRP)

    ones_g = jnp.ones((_GRP, 16), F32)
    zeros_np = jnp.zeros((_NPAD, 16), F32)

    b1r = b1.reshape(1, 16)
    b2r = b2.reshape(1, 16)
    wfct = Wfc.reshape(16, 1).T
    bfcr = bfc.reshape(1, 1)
    eye_k = jnp.eye(_K, dtype=F32)
    bd1 = jnp.kron(eye_k, Wc1)
    bd2t = jnp.kron(eye_k, Wc2).T
    bc1t = jnp.tile(bc1, _K).reshape(1, _K * _CF)
    bc2r = bc2.reshape(1, 1)

    degp = _sc_deg(dstp, ones_g, zeros_np)
    p0 = _tc_a1(x0p, W1)
    dinv16, g1 = _tc_prep(degp, p0)
    s1p = _sc_agg(g1, srcp, dstp, zeros_np)
    q0, g2 = _tc_comb1(s1p, p0, dinv16, b1r, W2)
    s2p = _sc_agg(g2, srcp, dstp, zeros_np)
    no0 = _tc_comb2(s2p, q0, dinv16, b2r, wfct, bfcr)

    nor = _tc_mlp_rest(xr, W1, b1r, W2, b2r, wfct, bfcr)

    node_out = jnp.concatenate(
        [no0[:, :_N], nor.reshape(b - 1, n)], axis=0)

    colpt = _tc_col(cols2, bd1, bc1t, bd2t, bc2r)
    colp3 = colpt.reshape(_K, b, n)
    out3 = _tc_softprod(node_out, colp3)
    logits = out3.transpose(1, 2, 0).reshape(b, n * _K)
    return logits


# speedup vs baseline: 225.7502x; 1.5362x over previous
import functools

import jax
import jax.numpy as jnp
from jax import lax
from jax.experimental import pallas as pl
from jax.experimental.pallas import tpu as pltpu
from jax.experimental.pallas import tpu_sc as plsc

F32 = jnp.float32

_B, _N, _F, _K, _CF, _E = 8, 10000, 128, 8, 16, 320000
_NPAD = 10048
_NPV = _NPAD // 8
_NW = 32
_GRP = 128
_GP = 80
_EPAD = _NW * _GP * _GRP
_KG = 4
_NCH = _GP // _KG


def _sc_mesh():
    return plsc.VectorSubcoreMesh(core_axis_name="c", subcore_axis_name="s")


_SC_PARAMS = pltpu.CompilerParams(use_tc_tiling_on_sc=False)


def _sc_deg(dstp, ones_g, zeros_np):
    @functools.partial(
        pl.kernel,
        out_type=jax.ShapeDtypeStruct((2, _NPAD, 16), F32),
        mesh=_sc_mesh(),
        compiler_params=_SC_PARAMS,
        scratch_types=[
            pltpu.VMEM((_GP, _GRP), jnp.int32),
            pltpu.VMEM((_GRP, 16), F32),
            pltpu.VMEM_SHARED((_NPAD, 16), F32),
            pltpu.SemaphoreType.DMA,
        ],
    )
    def k(dst_hbm, ones_hbm, zeros_hbm, out_hbm, didx_v, ones_v, shared,
          ssem):
        cid = lax.axis_index("c")
        sid = lax.axis_index("s")
        wid = sid * 2 + cid
        pltpu.sync_copy(dst_hbm.at[wid], didx_v)
        pltpu.sync_copy(ones_hbm, ones_v)

        @pl.when(sid == 0)
        def _():
            pltpu.sync_copy(zeros_hbm, shared)

        plsc.subcore_barrier()

        def issue(j, carry):
            pltpu.async_copy(ones_v, shared.at[didx_v.at[j]], ssem,
                             add=True)
            return carry

        lax.fori_loop(0, _GP, issue, 0)

        def drain(j, carry):
            pltpu.make_async_copy(ones_hbm, ones_v, ssem).wait()
            return carry

        lax.fori_loop(0, _GP, drain, 0)
        plsc.subcore_barrier()

        @pl.when(sid == 0)
        def _():
            pltpu.sync_copy(shared, out_hbm.at[cid])

    return k(dstp, ones_g, zeros_np)


def _sc_agg(g, srcp, dstp, zeros_np):
    @functools.partial(
        pl.kernel,
        out_type=jax.ShapeDtypeStruct((2, _NPAD, 16), F32),
        mesh=_sc_mesh(),
        compiler_params=_SC_PARAMS,
        scratch_types=[
            pltpu.VMEM((_GP, _GRP), jnp.int32),
            pltpu.VMEM((_GP, _GRP), jnp.int32),
            pltpu.VMEM((2, _KG * _GRP, 16), F32),
            pltpu.VMEM_SHARED((_NPAD, 16), F32),
            pltpu.SemaphoreType.DMA,
            pltpu.SemaphoreType.DMA,
        ],
    )
    def k(g_hbm, src_hbm, dst_hbm, zeros_hbm, out_hbm,
          sidx_v, didx_v, buf, shared, gsem, ssem):
        cid = lax.axis_index("c")
        sid = lax.axis_index("s")
        wid = sid * 2 + cid
        pltpu.sync_copy(src_hbm.at[wid], sidx_v)
        pltpu.sync_copy(dst_hbm.at[wid], didx_v)

        @pl.when(sid == 0)
        def _():
            pltpu.sync_copy(zeros_hbm, shared)

        plsc.subcore_barrier()

        for gi in range(_KG):
            pltpu.async_copy(g_hbm.at[sidx_v.at[gi]],
                             buf.at[0, pl.ds(gi * _GRP, _GRP)], gsem)

        def chunk(c, carry):
            s = lax.rem(c, 2)
            for gi in range(_KG):
                pltpu.make_async_copy(
                    g_hbm.at[pl.ds(0, _GRP)],
                    buf.at[s, pl.ds(gi * _GRP, _GRP)], gsem).wait()

            @pl.when(c >= 1)
            def _():
                for gi in range(_KG):
                    pltpu.make_async_copy(
                        g_hbm.at[pl.ds(0, _GRP)],
                        buf.at[1 - s, pl.ds(gi * _GRP, _GRP)], ssem).wait()

            @pl.when(c < _NCH - 1)
            def _():
                for gi in range(_KG):
                    pltpu.async_copy(
                        g_hbm.at[sidx_v.at[(c + 1) * _KG + gi]],
                        buf.at[1 - s, pl.ds(gi * _GRP, _GRP)], gsem)

            for gi in range(_KG):
                pltpu.async_copy(
                    buf.at[s, pl.ds(gi * _GRP, _GRP)],
                    shared.at[didx_v.at[c * _KG + gi]], ssem, add=True)
            return carry

        lax.fori_loop(0, _NCH, chunk, 0)

        for gi in range(_KG):
            pltpu.make_async_copy(
                g_hbm.at[pl.ds(0, _GRP)],
                buf.at[(_NCH - 1) % 2, pl.ds(gi * _GRP, _GRP)],
                ssem).wait()

        plsc.subcore_barrier()

        @pl.when(sid == 0)
        def _():
            pltpu.sync_copy(shared, out_hbm.at[cid])

    return k(g, srcp, dstp, zeros_np)


def _tc_a1prep(xv, wbig1, degpv):
    def body(x_ref, w_ref, d_ref, dinv_ref, g1_ref, p0_ref):
        p0 = jnp.dot(x_ref[...], w_ref[...], preferred_element_type=F32)
        cnt = d_ref[0] + d_ref[1]
        deg = cnt * float(_B) + 1.0
        dinv = lax.rsqrt(deg)
        dinv_ref[...] = dinv
        g1_ref[...] = p0 * dinv
        p0_ref[...] = p0

    return pl.pallas_call(
        body,
        grid=(1,),
        in_specs=[
            pl.BlockSpec((_NPV, 1024), lambda i: (0, 0)),
            pl.BlockSpec((1024, 128), lambda i: (0, 0)),
            pl.BlockSpec((2, _NPV, 128), lambda i: (0, 0, 0)),
        ],
        out_specs=(pl.BlockSpec((_NPV, 128), lambda i: (0, 0)),
                   pl.BlockSpec((_NPV, 128), lambda i: (0, 0)),
                   pl.BlockSpec((_NPV, 128), lambda i: (0, 0))),
        out_shape=(jax.ShapeDtypeStruct((_NPV, 128), F32),
                   jax.ShapeDtypeStruct((_NPV, 128), F32),
                   jax.ShapeDtypeStruct((_NPV, 128), F32)),
    )(xv, wbig1, degpv)


def _tc_comb1(s1pv, p0v, dinvv, b1v, w2big):
    def body(s_ref, p_ref, dinv_ref, b1_ref, w2_ref, q0_ref, g2_ref):
        dinv = dinv_ref[...]
        s = s_ref[0] + s_ref[1]
        h1 = jnp.maximum(
            s * (dinv * float(_B)) + dinv * dinv * p_ref[...] + b1_ref[...],
            0.0)
        q0 = jnp.dot(h1, w2_ref[...], preferred_element_type=F32)
        q0_ref[...] = q0
        g2_ref[...] = q0 * dinv

    return pl.pallas_call(
        body,
        out_shape=(jax.ShapeDtypeStruct((_NPV, 128), F32),
                   jax.ShapeDtypeStruct((_NPV, 128), F32)))(
            s1pv, p0v, dinvv, b1v, w2big)


def _tc_comb2(s2pv, q0v, dinvv, b2v, k2t, bfcr):
    def body(s_ref, q_ref, dinv_ref, b2_ref, kt_ref, bfc_ref, o_ref):
        dinv = dinv_ref[...]
        s = s_ref[0] + s_ref[1]
        h2 = jnp.maximum(
            s * (dinv * float(_B)) + dinv * dinv * q_ref[...] + b2_ref[...],
            0.0)
        o_ref[...] = lax.dot_general(
            kt_ref[...], h2, (((1,), (1,)), ((), ())),
            preferred_element_type=F32) + bfc_ref[...]

    return pl.pallas_call(
        body,
        out_shape=jax.ShapeDtypeStruct((8, _NPV), F32))(
            s2pv, q0v, dinvv, b2v, k2t, bfcr)


_RBLK = 2000


def _tc_mlp_rest(xall, w1, b1r, w2, b2r, wfct, bfcr):
    nblk = (_B - 1) * _N // _RBLK
    off = _N // _RBLK

    def body(x_ref, w1_ref, b1_ref, w2_ref, b2_ref, wt_ref, bfc_ref, o_ref):
        p = jnp.dot(x_ref[...], w1_ref[...], preferred_element_type=F32)
        h1 = jnp.maximum(p + b1_ref[...], 0.0)
        h2 = jnp.maximum(
            jnp.dot(h1, w2_ref[...], preferred_element_type=F32)
            + b2_ref[...], 0.0)
        o_ref[0] = lax.dot_general(
            wt_ref[...], h2, (((1,), (1,)), ((), ())),
            preferred_element_type=F32) + bfc_ref[...]

    return pl.pallas_call(
        body,
        grid=(nblk,),
        in_specs=[
            pl.BlockSpec((_RBLK, _F), lambda i: (i + off, 0)),
            pl.BlockSpec((_F, 16), lambda i: (0, 0)),
            pl.BlockSpec((1, 16), lambda i: (0, 0)),
            pl.BlockSpec((16, 16), lambda i: (0, 0)),
            pl.BlockSpec((1, 16), lambda i: (0, 0)),
            pl.BlockSpec((1, 16), lambda i: (0, 0)),
            pl.BlockSpec((1, 1), lambda i: (0, 0)),
        ],
        out_specs=pl.BlockSpec((1, 1, _RBLK), lambda i: (i, 0, 0)),
        out_shape=jax.ShapeDtypeStruct((nblk, 1, _RBLK), F32),
    )(xall, w1, b1r, w2, b2r, wfct, bfcr)


def _tc_col(colt, bd1t, bc1c, bd2t, bc2r):
    def body(c_ref, bd1_ref, bc1_ref, bd2t_ref, bc2_ref, o_ref):
        h = jnp.maximum(
            jnp.dot(bd1_ref[...], c_ref[0], preferred_element_type=F32)
            + bc1_ref[...], 0.0)
        ch = jnp.dot(bd2t_ref[...], h, preferred_element_type=F32) \
            + bc2_ref[...]
        m = jnp.max(ch, axis=0, keepdims=True)
        e = jnp.exp(ch - m)
        o_ref[0] = e / jnp.sum(e, axis=0, keepdims=True)

    kcf = _K * _CF
    return pl.pallas_call(
        body,
        grid=(_B,),
        in_specs=[
            pl.BlockSpec((1, kcf, _N), lambda i: (i, 0, 0)),
            pl.BlockSpec((kcf, kcf), lambda i: (0, 0)),
            pl.BlockSpec((kcf, 1), lambda i: (0, 0)),
            pl.BlockSpec((_K, kcf), lambda i: (0, 0)),
            pl.BlockSpec((1, 1), lambda i: (0, 0)),
        ],
        out_specs=pl.BlockSpec((1, _K, _N), lambda i: (i, 0, 0)),
        out_shape=jax.ShapeDtypeStruct((_B, _K, _N), F32),
    )(colt, bd1t, bc1c, bd2t, bc2r)


def _tc_softprod(node_out, colp):
    def body(no_ref, c_ref, o_ref):
        x = no_ref[0]
        m = jnp.max(x, axis=1, keepdims=True)
        e = jnp.exp(x - m)
        p = e / jnp.sum(e, axis=1, keepdims=True)
        prod = c_ref[0] * p
        a = prod.reshape(_K, _N // 16, 16)
        o_ref[0] = jnp.transpose(a, (1, 2, 0)).reshape(_N // 16, 16 * _K)

    return pl.pallas_call(
        body,
        grid=(_B,),
        in_specs=[
            pl.BlockSpec((1, 1, _N), lambda i: (i, 0, 0)),
            pl.BlockSpec((1, _K, _N), lambda i: (i, 0, 0)),
        ],
        out_specs=pl.BlockSpec((1, _N // 16, 16 * _K), lambda i: (i, 0, 0)),
        out_shape=jax.ShapeDtypeStruct((_B, _N // 16, 16 * _K), F32),
    )(node_out.reshape(_B, 1, _N), colp)


def kernel(node_features, col_features, edge_index, W1, b1, W2, b2,
           Wfc, bfc, Wc1, bc1, Wc2, bc2):
    b, n, f = node_features.shape
    assert (b, n, f) == (_B, _N, _F)
    e = edge_index.shape[1]
    assert e == _E

    xall = node_features.reshape(b * n, f)
    xv = xall.reshape(b * n // 8, 8 * f)
    colt = col_features.transpose(0, 2, 3, 1).reshape(b, _K * _CF, n)

    pad_e = _EPAD - _E
    srcp = jnp.concatenate(
        [edge_index[0], jnp.zeros((pad_e,), edge_index.dtype)]
    ).reshape(_NW, _GP, _GRP)
    dstp = jnp.concatenate(
        [edge_index[1], jnp.full((pad_e,), _N, edge_index.dtype)]
    ).reshape(_NW, _GP, _GRP)

    ones_g = jnp.ones((_GRP, 16), F32)
    zeros_np = jnp.zeros((_NPAD, 16), F32)

    eye_k = jnp.eye(8, dtype=F32)
    wbig1 = jnp.kron(eye_k, W1)
    w2big = jnp.kron(eye_k, W2)
    k2t = jnp.kron(eye_k, Wfc.reshape(16, 1).T)
    b1v = jnp.tile(b1, 8).reshape(1, 128)
    b2v = jnp.tile(b2, 8).reshape(1, 128)
    b1r = b1.reshape(1, 16)
    b2r = b2.reshape(1, 16)
    wfct = Wfc.reshape(16, 1).T
    bfcr = bfc.reshape(1, 1)
    bd1t = jnp.kron(eye_k, Wc1.T)
    bd2t = jnp.kron(eye_k, Wc2).T
    bc1c = jnp.tile(bc1, _K).reshape(_K * _CF, 1)
    bc2r = bc2.reshape(1, 1)

    degp = _sc_deg(dstp, ones_g, zeros_np)
    degpv = degp.reshape(2, _NPV, 128)
    dinvv, g1v, p0v = _tc_a1prep(xv, wbig1, degpv)
    s1p = _sc_agg(g1v.reshape(_NPAD, 16), srcp, dstp, zeros_np)
    q0v, g2v = _tc_comb1(s1p.reshape(2, _NPV, 128), p0v, dinvv, b1v, w2big)
    s2p = _sc_agg(g2v.reshape(_NPAD, 16), srcp, dstp, zeros_np)
    no8 = _tc_comb2(s2p.reshape(2, _NPV, 128), q0v, dinvv, b2v, k2t, bfcr)
    node0 = no8.T.reshape(1, _NPAD)[:, :_N]

    nor = _tc_mlp_rest(xall, W1, b1r, W2, b2r, wfct, bfcr)

    node_out = jnp.concatenate(
        [node0, nor.reshape(b - 1, n)], axis=0)

    colp = _tc_col(colt, bd1t, bc1c, bd2t, bc2r)
    out3 = _tc_softprod(node_out, colp)
    return out3.reshape(b, n * _K)


# speedup vs baseline: 284.5154x; 1.2603x over previous
import functools

import jax
import jax.numpy as jnp
from jax import lax
from jax.experimental import pallas as pl
from jax.experimental.pallas import tpu as pltpu
from jax.experimental.pallas import tpu_sc as plsc

F32 = jnp.float32

_B, _N, _F, _K, _CF, _E = 8, 10000, 128, 8, 16, 320000
_NPAD = 10048
_NPV = _NPAD // 8
_NW = 32
_GRP = 128
_GP = 80
_EPAD = _NW * _GP * _GRP
_KG = 4
_NCH = _GP // _KG


def _sc_mesh():
    return plsc.VectorSubcoreMesh(core_axis_name="c", subcore_axis_name="s")


_SC_PARAMS = pltpu.CompilerParams(use_tc_tiling_on_sc=False)


def _sc_deg(dstp, ones_g, zeros_np):
    @functools.partial(
        pl.kernel,
        out_type=jax.ShapeDtypeStruct((2, _NPAD, 16), F32),
        mesh=_sc_mesh(),
        compiler_params=_SC_PARAMS,
        scratch_types=[
            pltpu.VMEM((_GP, _GRP), jnp.int32),
            pltpu.VMEM((_GRP, 16), F32),
            pltpu.VMEM_SHARED((_NPAD, 16), F32),
            pltpu.SemaphoreType.DMA,
        ],
    )
    def k(dst_hbm, ones_hbm, zeros_hbm, out_hbm, didx_v, ones_v, shared,
          ssem):
        cid = lax.axis_index("c")
        sid = lax.axis_index("s")
        wid = sid * 2 + cid
        pltpu.sync_copy(dst_hbm.at[wid], didx_v)
        pltpu.sync_copy(ones_hbm, ones_v)

        @pl.when(sid == 0)
        def _():
            pltpu.sync_copy(zeros_hbm, shared)

        plsc.subcore_barrier()

        def issue(j, carry):
            pltpu.async_copy(ones_v, shared.at[didx_v.at[j]], ssem,
                             add=True)
            return carry

        lax.fori_loop(0, _GP, issue, 0)

        def drain(j, carry):
            pltpu.make_async_copy(ones_hbm, ones_v, ssem).wait()
            return carry

        lax.fori_loop(0, _GP, drain, 0)
        plsc.subcore_barrier()

        @pl.when(sid == 0)
        def _():
            pltpu.sync_copy(shared, out_hbm.at[cid])

    return k(dstp, ones_g, zeros_np)


def _sc_agg(g, srcp, dstp, zeros_np):
    @functools.partial(
        pl.kernel,
        out_type=jax.ShapeDtypeStruct((2, _NPAD, 16), F32),
        mesh=_sc_mesh(),
        compiler_params=_SC_PARAMS,
        scratch_types=[
            pltpu.VMEM((_GP, _GRP), jnp.int32),
            pltpu.VMEM((_GP, _GRP), jnp.int32),
            pltpu.VMEM((2, _KG * _GRP, 16), F32),
            pltpu.VMEM_SHARED((_NPAD, 16), F32),
            pltpu.SemaphoreType.DMA,
            pltpu.SemaphoreType.DMA,
        ],
    )
    def k(g_hbm, src_hbm, dst_hbm, zeros_hbm, out_hbm,
          sidx_v, didx_v, buf, shared, gsem, ssem):
        cid = lax.axis_index("c")
        sid = lax.axis_index("s")
        wid = sid * 2 + cid
        pltpu.sync_copy(src_hbm.at[wid], sidx_v)
        pltpu.sync_copy(dst_hbm.at[wid], didx_v)

        @pl.when(sid == 0)
        def _():
            pltpu.sync_copy(zeros_hbm, shared)

        plsc.subcore_barrier()

        for gi in range(_KG):
            pltpu.async_copy(g_hbm.at[sidx_v.at[gi]],
                             buf.at[0, pl.ds(gi * _GRP, _GRP)], gsem)

        def chunk(c, carry):
            s = lax.rem(c, 2)
            for gi in range(_KG):
                pltpu.make_async_copy(
                    g_hbm.at[pl.ds(0, _GRP)],
                    buf.at[s, pl.ds(gi * _GRP, _GRP)], gsem).wait()

            @pl.when(c >= 1)
            def _():
                for gi in range(_KG):
                    pltpu.make_async_copy(
                        g_hbm.at[pl.ds(0, _GRP)],
                        buf.at[1 - s, pl.ds(gi * _GRP, _GRP)], ssem).wait()

            @pl.when(c < _NCH - 1)
            def _():
                for gi in range(_KG):
                    pltpu.async_copy(
                        g_hbm.at[sidx_v.at[(c + 1) * _KG + gi]],
                        buf.at[1 - s, pl.ds(gi * _GRP, _GRP)], gsem)

            for gi in range(_KG):
                pltpu.async_copy(
                    buf.at[s, pl.ds(gi * _GRP, _GRP)],
                    shared.at[didx_v.at[c * _KG + gi]], ssem, add=True)
            return carry

        lax.fori_loop(0, _NCH, chunk, 0)

        for gi in range(_KG):
            pltpu.make_async_copy(
                g_hbm.at[pl.ds(0, _GRP)],
                buf.at[(_NCH - 1) % 2, pl.ds(gi * _GRP, _GRP)],
                ssem).wait()

        plsc.subcore_barrier()

        @pl.when(sid == 0)
        def _():
            pltpu.sync_copy(shared, out_hbm.at[cid])

    return k(g, srcp, dstp, zeros_np)


def _tc_a1prep(xv, wbig1, degpv):
    def body(x_ref, w_ref, d_ref, dinv_ref, g1_ref, p0_ref):
        p0 = jnp.dot(x_ref[...], w_ref[...], preferred_element_type=F32)
        cnt = d_ref[0] + d_ref[1]
        deg = cnt * float(_B) + 1.0
        dinv = lax.rsqrt(deg)
        dinv_ref[...] = dinv
        g1_ref[...] = p0 * dinv
        p0_ref[...] = p0

    return pl.pallas_call(
        body,
        grid=(1,),
        in_specs=[
            pl.BlockSpec((_NPV, 1024), lambda i: (0, 0)),
            pl.BlockSpec((1024, 128), lambda i: (0, 0)),
            pl.BlockSpec((2, _NPV, 128), lambda i: (0, 0, 0)),
        ],
        out_specs=(pl.BlockSpec((_NPV, 128), lambda i: (0, 0)),
                   pl.BlockSpec((_NPV, 128), lambda i: (0, 0)),
                   pl.BlockSpec((_NPV, 128), lambda i: (0, 0))),
        out_shape=(jax.ShapeDtypeStruct((_NPV, 128), F32),
                   jax.ShapeDtypeStruct((_NPV, 128), F32),
                   jax.ShapeDtypeStruct((_NPV, 128), F32)),
    )(xv, wbig1, degpv)


def _tc_comb1(s1pv, p0v, dinvv, b1v, w2big):
    def body(s_ref, p_ref, dinv_ref, b1_ref, w2_ref, q0_ref, g2_ref):
        dinv = dinv_ref[...]
        s = s_ref[0] + s_ref[1]
        h1 = jnp.maximum(
            s * (dinv * float(_B)) + dinv * dinv * p_ref[...] + b1_ref[...],
            0.0)
        q0 = jnp.dot(h1, w2_ref[...], preferred_element_type=F32)
        q0_ref[...] = q0
        g2_ref[...] = q0 * dinv

    return pl.pallas_call(
        body,
        out_shape=(jax.ShapeDtypeStruct((_NPV, 128), F32),
                   jax.ShapeDtypeStruct((_NPV, 128), F32)))(
            s1pv, p0v, dinvv, b1v, w2big)


def _tc_comb2(s2pv, q0v, dinvv, b2v, k2t, bfcr):
    def body(s_ref, q_ref, dinv_ref, b2_ref, kt_ref, bfc_ref, o_ref):
        dinv = dinv_ref[...]
        s = s_ref[0] + s_ref[1]
        h2 = jnp.maximum(
            s * (dinv * float(_B)) + dinv * dinv * q_ref[...] + b2_ref[...],
            0.0)
        o_ref[...] = lax.dot_general(
            kt_ref[...], h2, (((1,), (1,)), ((), ())),
            preferred_element_type=F32) + bfc_ref[...]

    return pl.pallas_call(
        body,
        out_shape=jax.ShapeDtypeStruct((8, _NPV), F32))(
            s2pv, q0v, dinvv, b2v, k2t, bfcr)


_RBLK = 2000


def _tc_mlp_rest(xall, w1, b1r, w2, b2r, wfct, bfcr):
    nblk = (_B - 1) * _N // _RBLK
    off = _N // _RBLK

    def body(x_ref, w1_ref, b1_ref, w2_ref, b2_ref, wt_ref, bfc_ref, o_ref):
        p = jnp.dot(x_ref[...], w1_ref[...], preferred_element_type=F32)
        h1 = jnp.maximum(p + b1_ref[...], 0.0)
        h2 = jnp.maximum(
            jnp.dot(h1, w2_ref[...], preferred_element_type=F32)
            + b2_ref[...], 0.0)
        o_ref[0] = lax.dot_general(
            wt_ref[...], h2, (((1,), (1,)), ((), ())),
            preferred_element_type=F32) + bfc_ref[...]

    return pl.pallas_call(
        body,
        grid=(nblk,),
        in_specs=[
            pl.BlockSpec((_RBLK, _F), lambda i: (i + off, 0)),
            pl.BlockSpec((_F, 16), lambda i: (0, 0)),
            pl.BlockSpec((1, 16), lambda i: (0, 0)),
            pl.BlockSpec((16, 16), lambda i: (0, 0)),
            pl.BlockSpec((1, 16), lambda i: (0, 0)),
            pl.BlockSpec((1, 16), lambda i: (0, 0)),
            pl.BlockSpec((1, 1), lambda i: (0, 0)),
        ],
        out_specs=pl.BlockSpec((1, 1, _RBLK), lambda i: (i, 0, 0)),
        out_shape=jax.ShapeDtypeStruct((nblk, 1, _RBLK), F32),
    )(xall, w1, b1r, w2, b2r, wfct, bfcr)


def _tc_col(colt, bd1t, bc1c, bd2t, bc2r):
    def body(c_ref, bd1_ref, bc1_ref, bd2t_ref, bc2_ref, o_ref):
        h = jnp.maximum(
            jnp.dot(bd1_ref[...], c_ref[0], preferred_element_type=F32)
            + bc1_ref[...], 0.0)
        ch = jnp.dot(bd2t_ref[...], h, preferred_element_type=F32) \
            + bc2_ref[...]
        m = jnp.max(ch, axis=0, keepdims=True)
        e = jnp.exp(ch - m)
        probs = e / jnp.sum(e, axis=0, keepdims=True)
        a = probs.reshape(_K, _N // 16, 16)
        o_ref[0] = jnp.transpose(a, (1, 2, 0)).reshape(_N // 16, 16 * _K)

    kcf = _K * _CF
    return pl.pallas_call(
        body,
        grid=(_B,),
        in_specs=[
            pl.BlockSpec((1, kcf, _N), lambda i: (i, 0, 0)),
            pl.BlockSpec((kcf, kcf), lambda i: (0, 0)),
            pl.BlockSpec((kcf, 1), lambda i: (0, 0)),
            pl.BlockSpec((_K, kcf), lambda i: (0, 0)),
            pl.BlockSpec((1, 1), lambda i: (0, 0)),
        ],
        out_specs=pl.BlockSpec((1, _N // 16, 16 * _K), lambda i: (i, 0, 0)),
        out_shape=jax.ShapeDtypeStruct((_B, _N // 16, 16 * _K), F32),
    )(colt, bd1t, bc1c, bd2t, bc2r)


def _softprod_body(no_ref, c_ref, o_ref):
    x = no_ref[0]
    m = jnp.max(x)
    e = jnp.exp(x - m)
    p = e / jnp.sum(e)
    pr = jnp.broadcast_to(p[:, :, None], (_N // 16, 16, _K))
    o_ref[0] = c_ref[0] * pr.reshape(_N // 16, 16 * _K)


def _tc_softprod_rest(node_out_rest, colp):
    body = _softprod_body
    return pl.pallas_call(
        body,
        grid=(_B - 1,),
        in_specs=[
            pl.BlockSpec((1, _N // 16, 16), lambda i: (i, 0, 0)),
            pl.BlockSpec((1, _N // 16, 16 * _K), lambda i: (i + 1, 0, 0)),
        ],
        out_specs=pl.BlockSpec((1, _N // 16, 16 * _K),
                               lambda i: (i + 1, 0, 0)),
        out_shape=jax.ShapeDtypeStruct((_B, _N // 16, 16 * _K), F32),
    )(node_out_rest, colp)


def _tc_softprod0(node0v, colp, prev):
    def body(no_ref, c_ref, p_ref, o_ref):
        x = no_ref[0]
        m = jnp.max(x)
        e = jnp.exp(x - m)
        p = e / jnp.sum(e)
        pr = jnp.broadcast_to(p[:, :, None], (_N // 16, 16, _K))
        o_ref[0] = c_ref[0] * pr.reshape(_N // 16, 16 * _K)

    return pl.pallas_call(
        body,
        grid=(1,),
        in_specs=[
            pl.BlockSpec((1, _N // 16, 16), lambda i: (0, 0, 0)),
            pl.BlockSpec((1, _N // 16, 16 * _K), lambda i: (0, 0, 0)),
            pl.BlockSpec((1, _N // 16, 16 * _K), lambda i: (0, 0, 0)),
        ],
        out_specs=pl.BlockSpec((1, _N // 16, 16 * _K), lambda i: (0, 0, 0)),
        out_shape=jax.ShapeDtypeStruct((_B, _N // 16, 16 * _K), F32),
        input_output_aliases={2: 0},
    )(node0v, colp, prev)


def kernel(node_features, col_features, edge_index, W1, b1, W2, b2,
           Wfc, bfc, Wc1, bc1, Wc2, bc2):
    b, n, f = node_features.shape
    assert (b, n, f) == (_B, _N, _F)
    e = edge_index.shape[1]
    assert e == _E

    xall = node_features.reshape(b * n, f)
    xv0 = jnp.pad(node_features[0].reshape(n // 8, 8 * f),
                  ((0, _NPV - n // 8), (0, 0)))
    colt = col_features.transpose(0, 2, 3, 1).reshape(b, _K * _CF, n)

    pad_e = _EPAD - _E
    srcp = jnp.concatenate(
        [edge_index[0], jnp.zeros((pad_e,), edge_index.dtype)]
    ).reshape(_NW, _GP, _GRP)
    dstp = jnp.concatenate(
        [edge_index[1], jnp.full((pad_e,), _N, edge_index.dtype)]
    ).reshape(_NW, _GP, _GRP)

    ones_g = jnp.ones((_GRP, 16), F32)
    zeros_np = jnp.zeros((_NPAD, 16), F32)

    eye_k = jnp.eye(8, dtype=F32)
    wbig1 = jnp.kron(eye_k, W1)
    w2big = jnp.kron(eye_k, W2)
    k2t = jnp.kron(eye_k, Wfc.reshape(16, 1).T)
    b1v = jnp.tile(b1, 8).reshape(1, 128)
    b2v = jnp.tile(b2, 8).reshape(1, 128)
    b1r = b1.reshape(1, 16)
    b2r = b2.reshape(1, 16)
    wfct = Wfc.reshape(16, 1).T
    bfcr = bfc.reshape(1, 1)
    bd1t = jnp.kron(eye_k, Wc1.T)
    bd2t = jnp.kron(eye_k, Wc2).T
    bc1c = jnp.tile(bc1, _K).reshape(_K * _CF, 1)
    bc2r = bc2.reshape(1, 1)

    nor = _tc_mlp_rest(xall, W1, b1r, W2, b2r, wfct, bfcr)
    colp = _tc_col(colt, bd1t, bc1c, bd2t, bc2r)
    nor_v = nor.reshape(b - 1, n // 16, 16)
    out7 = _tc_softprod_rest(nor_v, colp)

    degp = _sc_deg(dstp, ones_g, zeros_np)
    degpv = degp.reshape(2, _NPV, 128)
    dinvv, g1v, p0v = _tc_a1prep(xv0, wbig1, degpv)
    s1p = _sc_agg(g1v.reshape(_NPAD, 16), srcp, dstp, zeros_np)
    b1v_d = b1v + nor[0:1, 0:1, 0] * 0.0
    q0v, g2v = _tc_comb1(s1p.reshape(2, _NPV, 128), p0v, dinvv, b1v_d,
                         w2big)
    s2p = _sc_agg(g2v.reshape(_NPAD, 16), srcp, dstp, zeros_np)
    b2v_d = b2v + out7[1:2, 0:1, 0] * 0.0
    no8 = _tc_comb2(s2p.reshape(2, _NPV, 128), q0v, dinvv, b2v_d, k2t,
                    bfcr)
    node0v = no8.T.reshape(1, _NPAD)[:, :_N].reshape(1, n // 16, 16)

    out3 = _tc_softprod0(node0v, colp, out7)
    return out3.reshape(b, n * _K)
